# baseline (speedup 1.0000x reference)
import numpy as np

import jax
import jax.numpy as jnp
from jax.experimental import pallas as pl
from jax.experimental.pallas import tpu as pltpu

_FS = 15
_PAD = 7
_CO = 16
_G = 16


def _wavelet_kernel(x_ref, sel_ref, wb_ref, out_ref, xsp_ref, s_ref,
                    *, nb, nh, hp, w2, bimg):
    e0 = w2 + 8
    kk = nh * 120
    wm = wb_ref[...]
    for b in range(bimg):
        xs = x_ref[b, 0] + x_ref[b, 1] + x_ref[b, 2]

        xb = b % 2
        xsp_ref[xb, 0:8, :] = jnp.zeros_like(xsp_ref[xb, 0:8, :])
        xsp_ref[xb, hp - 8:hp, :] = jnp.zeros_like(xsp_ref[xb, hp - 8:hp, :])
        xsp_ref[xb, 8:hp - 8, :] = jnp.dot(
            xs.astype(jnp.bfloat16), sel_ref[...],
            preferred_element_type=jnp.float32)
        xsp = xsp_ref[xb]

        for dx in range(_FS):
            m = dx // 2
            if dx % 2 == 0:
                ev = xsp[:, m:m + w2]
                od = xsp[:, e0 + m:e0 + m + w2]
            else:
                ev = xsp[:, e0 + m:e0 + m + w2]
                od = xsp[:, m + 1:m + 1 + w2]
            piece = jnp.concatenate([ev, od], axis=1)
            s_ref[xb, :, dx * 8:(dx + 1) * 8, :] = (
                piece.reshape(hp // 8, 8, 2 * w2).astype(jnp.bfloat16))

        hg = _G // 2
        for g in range(nb):
            a = s_ref[xb, (_G // 8) * g:(_G // 8) * g + nh, :, :].reshape(
                kk, 2 * w2)
            y = jnp.dot(wm, a, preferred_element_type=jnp.float32)
            m2 = _CO * hg
            p = jnp.maximum(jnp.maximum(y[:m2], y[m2:]), 0.0)
            pc = jnp.maximum(p[:, :w2], p[:, w2:])
            out_ref[b, :, g * hg:(g + 1) * hg, :] = pc.reshape(_CO, hg, w2)


def kernel(x_nchw, weight):
    n, c, h, w = x_nchw.shape
    co = weight.shape[0]
    assert co == _CO and c == 3 and h % _G == 0 and w % 256 == 0
    h2, w2 = h // 2, w // 2
    nb = h // _G
    hp = h + 16
    nh = (_G + 14 + 7) // 8
    kk = nh * 120

    w0 = weight[:, 0, :, :].astype(jnp.bfloat16)

    hg = _G // 2
    k = np.arange(kk)
    s_loc = (k // 120) * 8 + (k % 8)
    dx = (k % 120) // 8
    oneh = np.zeros((2, _FS * _FS, hg * kk), np.float32)
    for p in range(2):
        for t in range(hg):
            dy = s_loc - 1 - (2 * t + p)
            valid = (dy >= 0) & (dy < _FS)
            f = np.clip(dy, 0, _FS - 1) * _FS + dx
            oneh[p, f[valid], t * kk + np.nonzero(valid)[0]] = 1.0
    w0flat = w0.reshape(co, _FS * _FS)
    wb = jnp.einsum("of,pfk->pok", w0flat,
                    jnp.asarray(oneh, jnp.bfloat16),
                    preferred_element_type=jnp.bfloat16)
    wb = wb.reshape(2, co, hg, kk).reshape(2 * co * hg, kk)

    j = np.arange(2 * (w2 + 8))
    tgt = np.where(j < w2 + 8, 2 * j - _PAD, 2 * (j - (w2 + 8)) - _PAD + 1)
    selp = jnp.asarray(
        (np.arange(w)[:, None] == tgt[None, :]).astype(np.float32),
        jnp.bfloat16)

    bimg = 1
    out = pl.pallas_call(
        lambda xr, cr, wr, orf, pr, sr: _wavelet_kernel(
            xr, cr, wr, orf, pr, sr, nb=nb, nh=nh, hp=hp, w2=w2, bimg=bimg),
        out_shape=jax.ShapeDtypeStruct((n, co, h2, w2), x_nchw.dtype),
        grid=(n // bimg,),
        in_specs=[
            pl.BlockSpec((bimg, c, h, w), lambda i: (i, 0, 0, 0)),
            pl.BlockSpec((w, 2 * (w2 + 8)), lambda i: (0, 0)),
            pl.BlockSpec((2 * co * hg, kk), lambda i: (0, 0)),
        ],
        out_specs=pl.BlockSpec((bimg, co, h2, w2), lambda i: (i, 0, 0, 0)),
        scratch_shapes=[pltpu.VMEM((2, hp, 2 * (w2 + 8)), jnp.float32),
                        pltpu.VMEM((2, hp // 8, _FS * 8, 2 * w2),
                                   jnp.bfloat16)],
        compiler_params=pltpu.CompilerParams(
            dimension_semantics=("parallel",),
            vmem_limit_bytes=48 * 1024 * 1024),
    )(x_nchw.astype(jnp.float32), selp, wb)
    return out


# speedup vs baseline: 1.1396x; 1.1396x over previous
import numpy as np

import jax
import jax.numpy as jnp
from jax.experimental import pallas as pl
from jax.experimental.pallas import tpu as pltpu

_FS = 15
_PAD = 7
_CO = 16
_G = 16


def _wavelet_kernel(x_ref, sel_ref, wb_ref, out_ref, xsp_ref, s_ref,
                    *, nb, nh, hp, w2, bimg):
    e0 = w2 + 8
    kk = nh * 120
    hh = hp - 16
    wm = wb_ref[...]

    xsum = [x_ref[b, 0] + x_ref[b, 1] + x_ref[b, 2] for b in range(bimg)]
    xcat = jnp.concatenate(xsum, axis=0).astype(jnp.bfloat16)
    packed = jnp.dot(xcat, sel_ref[...],
                     preferred_element_type=jnp.float32)
    for b in range(bimg):
        xb = b % 2
        xsp_ref[xb, 0:8, :] = jnp.zeros_like(xsp_ref[xb, 0:8, :])
        xsp_ref[xb, hp - 8:hp, :] = jnp.zeros_like(xsp_ref[xb, hp - 8:hp, :])
        xsp_ref[xb, 8:hp - 8, :] = packed[b * hh:(b + 1) * hh]

    for b in range(bimg):
        xb = b % 2
        xsp = xsp_ref[xb]

        for dx in range(_FS):
            m = dx // 2
            if dx % 2 == 0:
                ev = xsp[:, m:m + w2]
                od = xsp[:, e0 + m:e0 + m + w2]
            else:
                ev = xsp[:, e0 + m:e0 + m + w2]
                od = xsp[:, m + 1:m + 1 + w2]
            piece = jnp.concatenate([ev, od], axis=1)
            s_ref[xb, :, dx * 8:(dx + 1) * 8, :] = (
                piece.reshape(hp // 8, 8, 2 * w2).astype(jnp.bfloat16))

        hg = _G // 2
        for g in range(nb):
            a = s_ref[xb, (_G // 8) * g:(_G // 8) * g + nh, :, :].reshape(
                kk, 2 * w2)
            y = jnp.dot(wm, a, preferred_element_type=jnp.float32)
            m2 = _CO * hg
            p = jnp.maximum(jnp.maximum(y[:m2], y[m2:]), 0.0)
            pc = jnp.maximum(p[:, :w2], p[:, w2:])
            out_ref[b, :, g * hg:(g + 1) * hg, :] = pc.reshape(_CO, hg, w2)


def kernel(x_nchw, weight):
    n, c, h, w = x_nchw.shape
    co = weight.shape[0]
    assert co == _CO and c == 3 and h % _G == 0 and w % 256 == 0
    h2, w2 = h // 2, w // 2
    nb = h // _G
    hp = h + 16
    nh = (_G + 14 + 7) // 8
    kk = nh * 120

    w0 = weight[:, 0, :, :].astype(jnp.bfloat16)

    hg = _G // 2
    k = np.arange(kk)
    s_loc = (k // 120) * 8 + (k % 8)
    dx = (k % 120) // 8
    oneh = np.zeros((2, _FS * _FS, hg * kk), np.float32)
    for p in range(2):
        for t in range(hg):
            dy = s_loc - 1 - (2 * t + p)
            valid = (dy >= 0) & (dy < _FS)
            f = np.clip(dy, 0, _FS - 1) * _FS + dx
            oneh[p, f[valid], t * kk + np.nonzero(valid)[0]] = 1.0
    w0flat = w0.reshape(co, _FS * _FS)
    wb = jnp.einsum("of,pfk->pok", w0flat,
                    jnp.asarray(oneh, jnp.bfloat16),
                    preferred_element_type=jnp.bfloat16)
    wb = wb.reshape(2, co, hg, kk).reshape(2 * co * hg, kk)

    j = np.arange(2 * (w2 + 8))
    tgt = np.where(j < w2 + 8, 2 * j - _PAD, 2 * (j - (w2 + 8)) - _PAD + 1)
    selp = jnp.asarray(
        (np.arange(w)[:, None] == tgt[None, :]).astype(np.float32),
        jnp.bfloat16)

    bimg = 2 if n % 2 == 0 else 1
    out = pl.pallas_call(
        lambda xr, cr, wr, orf, pr, sr: _wavelet_kernel(
            xr, cr, wr, orf, pr, sr, nb=nb, nh=nh, hp=hp, w2=w2, bimg=bimg),
        out_shape=jax.ShapeDtypeStruct((n, co, h2, w2), x_nchw.dtype),
        grid=(n // bimg,),
        in_specs=[
            pl.BlockSpec((bimg, c, h, w), lambda i: (i, 0, 0, 0)),
            pl.BlockSpec((w, 2 * (w2 + 8)), lambda i: (0, 0)),
            pl.BlockSpec((2 * co * hg, kk), lambda i: (0, 0)),
        ],
        out_specs=pl.BlockSpec((bimg, co, h2, w2), lambda i: (i, 0, 0, 0)),
        scratch_shapes=[pltpu.VMEM((2, hp, 2 * (w2 + 8)), jnp.float32),
                        pltpu.VMEM((2, hp // 8, _FS * 8, 2 * w2),
                                   jnp.bfloat16)],
        compiler_params=pltpu.CompilerParams(
            dimension_semantics=("parallel",),
            vmem_limit_bytes=48 * 1024 * 1024),
    )(x_nchw.astype(jnp.float32), selp, wb)
    return out


# speedup vs baseline: 1.1588x; 1.0169x over previous
import numpy as np

import jax
import jax.numpy as jnp
from jax.experimental import pallas as pl
from jax.experimental.pallas import tpu as pltpu

_FS = 15
_PAD = 7
_CO = 16
_G = 16


def _wavelet_kernel(x_ref, sel_ref, wb_ref, out_ref, xsp_ref, s_ref,
                    *, nb, nh, hp, w2, bimg):
    e0 = w2 + 8
    kk = nh * 120
    hh = hp - 16
    wm = wb_ref[...]

    xsum = [x_ref[b, 0] + x_ref[b, 1] + x_ref[b, 2] for b in range(bimg)]
    xcat = jnp.concatenate(xsum, axis=0).astype(jnp.bfloat16)
    packed = jnp.dot(xcat, sel_ref[...],
                     preferred_element_type=jnp.float32)
    for b in range(bimg):
        xb = b % 2
        xsp_ref[xb, 0:8, :] = jnp.zeros_like(xsp_ref[xb, 0:8, :])
        xsp_ref[xb, hp - 8:hp, :] = jnp.zeros_like(xsp_ref[xb, hp - 8:hp, :])
        xsp_ref[xb, 8:hp - 8, :] = packed[b * hh:(b + 1) * hh]

    for b in range(bimg):
        xb = b % 2
        xsp = xsp_ref[xb]

        for dx in range(_FS):
            m = dx // 2
            if dx % 2 == 0:
                ev = xsp[:, m:m + w2]
                od = xsp[:, e0 + m:e0 + m + w2]
            else:
                ev = xsp[:, e0 + m:e0 + m + w2]
                od = xsp[:, m + 1:m + 1 + w2]
            piece = jnp.concatenate([ev, od], axis=1)
            s_ref[xb, :, dx * 8:(dx + 1) * 8, :] = (
                piece.reshape(hp // 8, 8, 2 * w2).astype(jnp.bfloat16))

        hg = _G // 2
        for g in range(nb):
            a = s_ref[xb, (_G // 8) * g:(_G // 8) * g + nh, :, :].reshape(
                kk, 2 * w2)
            y = jnp.dot(wm, a, preferred_element_type=jnp.float32)
            m2 = _CO * hg
            p = jnp.maximum(jnp.maximum(y[:m2], y[m2:]), 0.0)
            pc = jnp.maximum(p[:, :w2], p[:, w2:])
            out_ref[b, :, g * hg:(g + 1) * hg, :] = pc.reshape(_CO, hg, w2)


def kernel(x_nchw, weight):
    n, c, h, w = x_nchw.shape
    co = weight.shape[0]
    assert co == _CO and c == 3 and h % _G == 0 and w % 256 == 0
    h2, w2 = h // 2, w // 2
    nb = h // _G
    hp = h + 16
    nh = (_G + 14 + 7) // 8
    kk = nh * 120

    w0 = weight[:, 0, :, :].astype(jnp.bfloat16)

    hg = _G // 2
    k = np.arange(kk)
    s_loc = (k // 120) * 8 + (k % 8)
    dx = (k % 120) // 8
    oneh = np.zeros((2, _FS * _FS, hg * kk), np.float32)
    for p in range(2):
        for t in range(hg):
            dy = s_loc - 1 - (2 * t + p)
            valid = (dy >= 0) & (dy < _FS)
            f = np.clip(dy, 0, _FS - 1) * _FS + dx
            oneh[p, f[valid], t * kk + np.nonzero(valid)[0]] = 1.0
    w0flat = w0.reshape(co, _FS * _FS)
    wb = jnp.einsum("of,pfk->pok", w0flat,
                    jnp.asarray(oneh, jnp.bfloat16),
                    preferred_element_type=jnp.bfloat16)
    wb = wb.reshape(2, co, hg, kk).reshape(2 * co * hg, kk)

    j = np.arange(2 * (w2 + 8))
    tgt = np.where(j < w2 + 8, 2 * j - _PAD, 2 * (j - (w2 + 8)) - _PAD + 1)
    selp = jnp.asarray(
        (np.arange(w)[:, None] == tgt[None, :]).astype(np.float32),
        jnp.bfloat16)

    bimg = 4 if n % 4 == 0 else 1
    out = pl.pallas_call(
        lambda xr, cr, wr, orf, pr, sr: _wavelet_kernel(
            xr, cr, wr, orf, pr, sr, nb=nb, nh=nh, hp=hp, w2=w2, bimg=bimg),
        out_shape=jax.ShapeDtypeStruct((n, co, h2, w2), x_nchw.dtype),
        grid=(n // bimg,),
        in_specs=[
            pl.BlockSpec((bimg, c, h, w), lambda i: (i, 0, 0, 0)),
            pl.BlockSpec((w, 2 * (w2 + 8)), lambda i: (0, 0)),
            pl.BlockSpec((2 * co * hg, kk), lambda i: (0, 0)),
        ],
        out_specs=pl.BlockSpec((bimg, co, h2, w2), lambda i: (i, 0, 0, 0)),
        scratch_shapes=[pltpu.VMEM((2, hp, 2 * (w2 + 8)), jnp.float32),
                        pltpu.VMEM((2, hp // 8, _FS * 8, 2 * w2),
                                   jnp.bfloat16)],
        compiler_params=pltpu.CompilerParams(
            dimension_semantics=("parallel",),
            vmem_limit_bytes=48 * 1024 * 1024),
    )(x_nchw.astype(jnp.float32), selp, wb)
    return out


# speedup vs baseline: 1.2014x; 1.0367x over previous
import numpy as np

import jax
import jax.numpy as jnp
from jax.experimental import pallas as pl
from jax.experimental.pallas import tpu as pltpu

_FS = 15
_PAD = 7
_CO = 16
_G = 16


def _wavelet_kernel(x_ref, sel_ref, wb_ref, out_ref, xsp_ref, s_ref,
                    *, nb, nh, hp, w2, bimg):
    e0 = w2 + 8
    kk = nh * 120
    hh = hp - 16
    wm = wb_ref[...]

    xsum = [x_ref[b, 0] + x_ref[b, 1] + x_ref[b, 2] for b in range(bimg)]
    xcat = jnp.concatenate(xsum, axis=0).astype(jnp.bfloat16)
    packed = jnp.dot(xcat, sel_ref[...],
                     preferred_element_type=jnp.float32)
    for b in range(bimg):
        xsp_ref[b, 0:8, :] = jnp.zeros_like(xsp_ref[b, 0:8, :])
        xsp_ref[b, hp - 8:hp, :] = jnp.zeros_like(xsp_ref[b, hp - 8:hp, :])
        xsp_ref[b, 8:hp - 8, :] = packed[b * hh:(b + 1) * hh]

    for b in range(bimg):
        xb = b % 2
        xsp = xsp_ref[b]

        for dx in range(_FS):
            m = dx // 2
            if dx % 2 == 0:
                ev = xsp[:, m:m + w2]
                od = xsp[:, e0 + m:e0 + m + w2]
            else:
                ev = xsp[:, e0 + m:e0 + m + w2]
                od = xsp[:, m + 1:m + 1 + w2]
            piece = jnp.concatenate([ev, od], axis=1)
            s_ref[xb, :, dx * 8:(dx + 1) * 8, :] = (
                piece.reshape(hp // 8, 8, 2 * w2).astype(jnp.bfloat16))

        hg = _G // 2
        for g in range(nb):
            a = s_ref[xb, (_G // 8) * g:(_G // 8) * g + nh, :, :].reshape(
                kk, 2 * w2)
            y = jnp.dot(wm, a, preferred_element_type=jnp.float32)
            m2 = _CO * hg
            p = jnp.maximum(jnp.maximum(y[:m2], y[m2:]), 0.0)
            pc = jnp.maximum(p[:, :w2], p[:, w2:])
            out_ref[b, :, g * hg:(g + 1) * hg, :] = pc.reshape(_CO, hg, w2)


def kernel(x_nchw, weight):
    n, c, h, w = x_nchw.shape
    co = weight.shape[0]
    assert co == _CO and c == 3 and h % _G == 0 and w % 256 == 0
    h2, w2 = h // 2, w // 2
    nb = h // _G
    hp = h + 16
    nh = (_G + 14 + 7) // 8
    kk = nh * 120

    w0 = weight[:, 0, :, :].astype(jnp.bfloat16)

    hg = _G // 2
    k = np.arange(kk)
    s_loc = (k // 120) * 8 + (k % 8)
    dx = (k % 120) // 8
    oneh = np.zeros((2, _FS * _FS, hg * kk), np.float32)
    for p in range(2):
        for t in range(hg):
            dy = s_loc - 1 - (2 * t + p)
            valid = (dy >= 0) & (dy < _FS)
            f = np.clip(dy, 0, _FS - 1) * _FS + dx
            oneh[p, f[valid], t * kk + np.nonzero(valid)[0]] = 1.0
    w0flat = w0.reshape(co, _FS * _FS)
    wb = jnp.einsum("of,pfk->pok", w0flat,
                    jnp.asarray(oneh, jnp.bfloat16),
                    preferred_element_type=jnp.bfloat16)
    wb = wb.reshape(2, co, hg, kk).reshape(2 * co * hg, kk)

    j = np.arange(2 * (w2 + 8))
    tgt = np.where(j < w2 + 8, 2 * j - _PAD, 2 * (j - (w2 + 8)) - _PAD + 1)
    selp = jnp.asarray(
        (np.arange(w)[:, None] == tgt[None, :]).astype(np.float32),
        jnp.bfloat16)

    bimg = 4 if n % 4 == 0 else 1
    out = pl.pallas_call(
        lambda xr, cr, wr, orf, pr, sr: _wavelet_kernel(
            xr, cr, wr, orf, pr, sr, nb=nb, nh=nh, hp=hp, w2=w2, bimg=bimg),
        out_shape=jax.ShapeDtypeStruct((n, co, h2, w2), x_nchw.dtype),
        grid=(n // bimg,),
        in_specs=[
            pl.BlockSpec((bimg, c, h, w), lambda i: (i, 0, 0, 0)),
            pl.BlockSpec((w, 2 * (w2 + 8)), lambda i: (0, 0)),
            pl.BlockSpec((2 * co * hg, kk), lambda i: (0, 0)),
        ],
        out_specs=pl.BlockSpec((bimg, co, h2, w2), lambda i: (i, 0, 0, 0)),
        scratch_shapes=[pltpu.VMEM((bimg, hp, 2 * (w2 + 8)), jnp.float32),
                        pltpu.VMEM((2, hp // 8, _FS * 8, 2 * w2),
                                   jnp.bfloat16)],
        compiler_params=pltpu.CompilerParams(
            dimension_semantics=("parallel",),
            vmem_limit_bytes=48 * 1024 * 1024),
    )(x_nchw.astype(jnp.float32), selp, wb)
    return out
